# Initial kernel scaffold; baseline (speedup 1.0000x reference)
import jax
import jax.numpy as jnp
from jax.experimental import pallas as pl


def kernel(x, w1, b1, s1, t1, w2, b2, s2, t2, w3, b3, s3, t3, w4, b4, s4, t4, w5, b5, s5, t5, w6, b6i):
    raise NotImplementedError("write your pallas kernel here")


# speedup vs baseline: 1.5628x; 1.5628x over previous
import functools

import jax
import jax.numpy as jnp
from jax.experimental import pallas as pl
from jax.experimental.pallas import tpu as pltpu


def _fused_stn_kernel(
    x_ref,
    w1_ref, b1_ref,
    w2_ref, b2_ref,
    w3_ref, b3_ref, s3_ref, t3_ref,
    w4_ref, b4_ref,
    w5_ref, b5_ref,
    w6_ref, b6_ref,
    o_ref,
    *, n_chunk: int, n_valid: int,
):
    xb = x_ref[0]
    k_dim, n_pad = xb.shape
    c_hidden = w3_ref.shape[0]

    hmax = jnp.full((c_hidden, 1), -jnp.inf, jnp.float32)
    for c in range(n_pad // n_chunk):
        xc = xb[:, c * n_chunk:(c + 1) * n_chunk].astype(jnp.bfloat16)
        z1 = jnp.dot(w1_ref[...], xc,
                     preferred_element_type=jnp.float32) + b1_ref[...]
        h1 = jnp.maximum(z1, 0.0).astype(jnp.bfloat16)
        z2 = jnp.dot(w2_ref[...], h1,
                     preferred_element_type=jnp.float32) + b2_ref[...]
        h2 = jnp.maximum(z2, 0.0).astype(jnp.bfloat16)
        z3 = jnp.dot(w3_ref[...], h2,
                     preferred_element_type=jnp.float32) + b3_ref[...]
        a3 = jnp.maximum(z3, 0.0) * s3_ref[...] + t3_ref[...]
        if c * n_chunk + n_chunk > n_valid:
            col = c * n_chunk + jax.lax.broadcasted_iota(
                jnp.int32, (1, n_chunk), 1)
            a3 = jnp.where(col < n_valid, a3, -jnp.inf)
        hmax = jnp.maximum(hmax, jnp.max(a3, axis=-1, keepdims=True))

    g = hmax.astype(jnp.bfloat16)
    f4 = jnp.maximum(
        jnp.dot(w4_ref[...], g, preferred_element_type=jnp.float32)
        + b4_ref[...], 0.0).astype(jnp.bfloat16)
    f5 = jnp.maximum(
        jnp.dot(w5_ref[...], f4, preferred_element_type=jnp.float32)
        + b5_ref[...], 0.0).astype(jnp.bfloat16)
    m = jnp.dot(w6_ref[...], f5,
                preferred_element_type=jnp.float32) + b6_ref[...]
    mat = m.reshape(k_dim, k_dim)

    o_ref[0] = jnp.dot(mat, xb, preferred_element_type=jnp.float32)


def kernel(x,
           w1, b1, s1, t1,
           w2, b2, s2, t2,
           w3, b3, s3, t3,
           w4, b4, s4, t4,
           w5, b5, s5, t5,
           w6, b6i):
    B, K, N = x.shape
    bf = jnp.bfloat16

    w1b = w1.astype(bf)
    w2f = (w2 * s1.reshape(1, -1)).astype(bf)
    b2f = b2 + w2 @ t1
    w3f = (w3 * s2.reshape(1, -1)).astype(bf)
    b3f = b3 + w3 @ t2
    w4t = w4.T.astype(bf)
    b4t = b4.reshape(-1, 1)
    w5t = (w5 * s4.reshape(-1, 1)).T.astype(bf)
    b5t = (b5 + t4 @ w5).reshape(-1, 1)
    w6t = (w6 * s5.reshape(-1, 1)).T.astype(bf)
    b6t = (b6i + t5 @ w6).reshape(-1, 1)

    n_chunk = min(512, ((N + 127) // 128) * 128)
    n_pad = ((N + n_chunk - 1) // n_chunk) * n_chunk
    x_pad = jnp.pad(x, ((0, 0), (0, 0), (0, n_pad - N))) if n_pad != N else x

    body = functools.partial(_fused_stn_kernel, n_chunk=n_chunk, n_valid=N)
    params = (w1b, b1, w2f, b2f, w3f, b3f, s3, t3, w4t, b4t, w5t, b5t, w6t, b6t)
    out_pad = pl.pallas_call(
        body,
        out_shape=jax.ShapeDtypeStruct((B, K, n_pad), jnp.float32),
        grid=(B,),
        in_specs=[pl.BlockSpec((1, K, n_pad), lambda b: (b, 0, 0))]
        + [pl.BlockSpec(p.shape, lambda b: (0,) * p.ndim) for p in params],
        out_specs=pl.BlockSpec((1, K, n_pad), lambda b: (b, 0, 0)),
        compiler_params=pltpu.CompilerParams(
            dimension_semantics=("parallel",)),
    )(x_pad, *params)

    return out_pad[:, :, :N] if n_pad != N else out_pad


# speedup vs baseline: 2.8394x; 1.8169x over previous
import functools

import jax
import jax.numpy as jnp
from jax.experimental import pallas as pl
from jax.experimental.pallas import tpu as pltpu


def _fused_stn_kernel(
    x_ref,
    w1_ref, b1_ref,
    w2_ref, b2_ref,
    w3_ref, b3_ref, s3_ref,
    w4_ref, b4_ref,
    w5_ref, b5_ref,
    w6_ref, b6_ref,
    o_ref,
    *, n_chunk: int, n_valid: int, g_batch: int,
):
    bf = jnp.bfloat16
    k_dim = x_ref.shape[1]
    n_pad = x_ref.shape[2]
    n_chunks = n_pad // n_chunk

    g_cols = []
    for g in range(g_batch):
        xbf = x_ref[g].astype(bf)
        z1 = jnp.dot(w1_ref[...], xbf,
                     preferred_element_type=jnp.float32) + b1_ref[...]
        h1 = jnp.maximum(z1, 0.0).astype(bf)
        z2 = jnp.dot(w2_ref[...], h1,
                     preferred_element_type=jnp.float32) + b2_ref[...]
        h2 = jnp.maximum(z2, 0.0).astype(bf)

        zmax = None
        zmin = None
        for c in range(n_chunks):
            hc = h2[:, c * n_chunk:(c + 1) * n_chunk]
            z3 = jnp.dot(w3_ref[...], hc,
                         preferred_element_type=jnp.float32) + b3_ref[...]
            if c * n_chunk + n_chunk > n_valid:
                col = c * n_chunk + jax.lax.broadcasted_iota(
                    jnp.int32, (1, n_chunk), 1)
                valid = col < n_valid
                cmax = jnp.max(jnp.where(valid, z3, -jnp.inf),
                               axis=-1, keepdims=True)
                cmin = jnp.min(jnp.where(valid, z3, jnp.inf),
                               axis=-1, keepdims=True)
            else:
                cmax = jnp.max(z3, axis=-1, keepdims=True)
                cmin = jnp.min(z3, axis=-1, keepdims=True)
            zmax = cmax if zmax is None else jnp.maximum(zmax, cmax)
            zmin = cmin if zmin is None else jnp.minimum(zmin, cmin)

        r = jnp.maximum(jnp.where(s3_ref[...] >= 0.0, zmax, zmin), 0.0)
        g_cols.append(r.astype(bf))

    gmat = (g_cols[0] if g_batch == 1
            else jnp.concatenate(g_cols, axis=1))
    f4 = jnp.maximum(
        jnp.dot(w4_ref[...], gmat, preferred_element_type=jnp.float32)
        + b4_ref[...], 0.0).astype(bf)
    f5 = jnp.maximum(
        jnp.dot(w5_ref[...], f4, preferred_element_type=jnp.float32)
        + b5_ref[...], 0.0).astype(bf)
    m = jnp.dot(w6_ref[...], f5,
                preferred_element_type=jnp.float32) + b6_ref[...]

    for g in range(g_batch):
        mat = m[:, g:g + 1].reshape(k_dim, k_dim).astype(bf)
        xbf = x_ref[g].astype(bf)
        o_ref[g] = jnp.dot(mat, xbf, preferred_element_type=jnp.float32)


def kernel(x,
           w1, b1, s1, t1,
           w2, b2, s2, t2,
           w3, b3, s3, t3,
           w4, b4, s4, t4,
           w5, b5, s5, t5,
           w6, b6i):
    B, K, N = x.shape
    bf = jnp.bfloat16

    w1b = w1.astype(bf)
    w2f = (w2 * s1.reshape(1, -1)).astype(bf)
    b2f = b2 + w2 @ t1
    w3f = (w3 * s2.reshape(1, -1)).astype(bf)
    b3f = b3 + w3 @ t2
    w4t = w4.T
    w4f = (w4t * s3.reshape(1, -1)).astype(bf)
    b4f = (b4.reshape(-1, 1) + w4t @ t3)
    w5t = (w5 * s4.reshape(-1, 1)).T.astype(bf)
    b5t = (b5 + t4 @ w5).reshape(-1, 1)
    w6t = (w6 * s5.reshape(-1, 1)).T.astype(bf)
    b6t = (b6i + t5 @ w6).reshape(-1, 1)

    g_batch = 4
    while B % g_batch:
        g_batch //= 2
    n_chunk = min(1024, ((N + 127) // 128) * 128)
    n_pad = ((N + n_chunk - 1) // n_chunk) * n_chunk
    x_pad = jnp.pad(x, ((0, 0), (0, 0), (0, n_pad - N))) if n_pad != N else x

    body = functools.partial(_fused_stn_kernel, n_chunk=n_chunk,
                             n_valid=N, g_batch=g_batch)
    params = (w1b, b1, w2f, b2f, w3f, b3f, s3, w4f, b4f, w5t, b5t, w6t, b6t)
    out_pad = pl.pallas_call(
        body,
        out_shape=jax.ShapeDtypeStruct((B, K, n_pad), jnp.float32),
        grid=(B // g_batch,),
        in_specs=[pl.BlockSpec((g_batch, K, n_pad), lambda b: (b, 0, 0))]
        + [pl.BlockSpec(p.shape, lambda b: (0,) * p.ndim) for p in params],
        out_specs=pl.BlockSpec((g_batch, K, n_pad), lambda b: (b, 0, 0)),
        compiler_params=pltpu.CompilerParams(
            dimension_semantics=("parallel",)),
    )(x_pad, *params)

    return out_pad[:, :, :N] if n_pad != N else out_pad
